# Initial kernel scaffold; baseline (speedup 1.0000x reference)
import jax
import jax.numpy as jnp
from jax.experimental import pallas as pl


def kernel(x, w1, w2):
    raise NotImplementedError("write your pallas kernel here")


# speedup vs baseline: 1.2325x; 1.2325x over previous
import functools

import jax
import jax.numpy as jnp
from jax import lax
from jax.experimental import pallas as pl
from jax.experimental.pallas import tpu as pltpu

_K = 3
_P = 1


def _build_acc2(x_blk, w1c, w2c, geom):
    (c_in, c_mid, h, w, hp, w1p, m1, m2) = geom
    bf16 = jnp.bfloat16
    xr = jnp.maximum(x_blk, 0.0).astype(bf16)
    zr = jnp.zeros((c_in, _P, w), bf16)
    zc = jnp.zeros((c_in, hp, 2 * _P), bf16)
    xp = jnp.concatenate(
        [zc, jnp.concatenate([zr, xr, zr], axis=1), zc], axis=2)
    xf = xp.reshape(c_in, hp * w1p)
    x3 = jnp.concatenate([xf[:, t * w1p:t * w1p + m1] for t in range(_K)],
                         axis=0)
    acc1 = jnp.dot(w1c, x3, preferred_element_type=jnp.float32)
    y1p = jnp.concatenate(
        [jnp.zeros((c_mid, _P * w1p), bf16), acc1.astype(bf16),
         jnp.zeros((c_mid, m2 + _K - 1 - _P * w1p - m1), bf16)], axis=1)
    y3 = jnp.concatenate([y1p[:, t:t + m2] for t in range(_K)], axis=0)
    return jnp.dot(w2c, y3, preferred_element_type=jnp.float32)


def _stats_kernel(x_ref, w1c_ref, w2c_ref, mask_ref, psum_ref, psq_ref, *,
                  geom):
    acc2 = _build_acc2(x_ref[0], w1c_ref[...], w2c_ref[...], geom)
    masked = acc2 * mask_ref[...]
    psum_ref[0] = jnp.sum(masked, axis=1, keepdims=True)
    psq_ref[0] = jnp.sum(masked * masked, axis=1, keepdims=True)


def _out_kernel(x_ref, w1c_ref, w2s_ref, bias_ref, o_ref, *, geom):
    (c_in, c_mid, h, w, hp, w1p, m1, m2) = geom
    c_out = o_ref.shape[1]
    h2 = o_ref.shape[2]
    w2o = o_ref.shape[3]
    acc2 = _build_acc2(x_ref[0], w1c_ref[...], w2s_ref[...], geom)
    y = acc2.reshape(c_out, h2, w1p)[:, :, :w2o] + bias_ref[...].reshape(
        c_out, 1, 1)
    o_ref[0] = y.astype(o_ref.dtype)


def kernel(x, w1, w2):
    n, c_in, h, w = x.shape
    c_mid = w1.shape[0]
    c_out = w2.shape[0]
    k, p = _K, _P
    eps = 1e-5
    hp = h + 2 * p
    h1 = hp - k + 1
    w1p = w + 4 * p
    h2 = h1 + 2 * p
    w2o = w1p - k + 1
    m1 = h1 * w1p
    m2 = h2 * w1p
    geom = (c_in, c_mid, h, w, hp, w1p, m1, m2)

    f32 = jnp.float32
    bf16 = jnp.bfloat16
    w1c = jnp.concatenate([w1[:, :, t, 0] for t in range(k)],
                          axis=1).astype(bf16)
    w2c_f = jnp.concatenate([w2[:, :, 0, t] for t in range(k)],
                            axis=1).astype(f32)
    mask = (jnp.arange(m2, dtype=jnp.int32) % w1p < w2o).astype(f32).reshape(
        1, m2)

    psum, psq = pl.pallas_call(
        functools.partial(_stats_kernel, geom=geom),
        out_shape=(
            jax.ShapeDtypeStruct((n, c_out, 1), f32),
            jax.ShapeDtypeStruct((n, c_out, 1), f32),
        ),
        grid=(n,),
        in_specs=[
            pl.BlockSpec((1, c_in, h, w), lambda i: (i, 0, 0, 0)),
            pl.BlockSpec((c_mid, k * c_in), lambda i: (0, 0)),
            pl.BlockSpec((c_out, k * c_mid), lambda i: (0, 0)),
            pl.BlockSpec((1, m2), lambda i: (0, 0)),
        ],
        out_specs=(
            pl.BlockSpec((1, c_out, 1), lambda i: (i, 0, 0)),
            pl.BlockSpec((1, c_out, 1), lambda i: (i, 0, 0)),
        ),
        compiler_params=pltpu.CompilerParams(
            dimension_semantics=("parallel",)),
    )(x, w1c, w2c_f.astype(bf16), mask)

    cnt = jnp.float32(n * h2 * w2o)
    mean = jnp.sum(psum, axis=0) / cnt
    var = jnp.sum(psq, axis=0) / cnt - mean * mean
    rstd = lax.rsqrt(var + eps)
    w2s = (w2c_f * rstd).astype(bf16)
    bias = (-mean * rstd)

    out = pl.pallas_call(
        functools.partial(_out_kernel, geom=geom),
        out_shape=jax.ShapeDtypeStruct((n, c_out, h2, w2o), x.dtype),
        grid=(n,),
        in_specs=[
            pl.BlockSpec((1, c_in, h, w), lambda i: (i, 0, 0, 0)),
            pl.BlockSpec((c_mid, k * c_in), lambda i: (0, 0)),
            pl.BlockSpec((c_out, k * c_mid), lambda i: (0, 0)),
            pl.BlockSpec((c_out, 1), lambda i: (0, 0)),
        ],
        out_specs=pl.BlockSpec((1, c_out, h2, w2o), lambda i: (i, 0, 0, 0)),
        compiler_params=pltpu.CompilerParams(
            dimension_semantics=("parallel",)),
    )(x, w1c, w2s, bias)

    return out


# speedup vs baseline: 1.6894x; 1.3707x over previous
import functools

import jax
import jax.numpy as jnp
from jax import lax
from jax.experimental import pallas as pl
from jax.experimental.pallas import tpu as pltpu


def _conv_core(x_blk, w1c, w2c, c_in, c_mid, w34, ell):
    bf16 = jnp.bfloat16
    xr = jnp.maximum(x_blk, 0.0).astype(bf16)
    xpad = jnp.pad(xr, ((0, 0), (1, 1), (1, 1)))
    x_f = xpad.reshape(c_in, ell)
    z = jnp.zeros((c_in, w34), bf16)
    span = ell - 2 * w34
    t0 = jnp.concatenate([z, x_f[:, :span], z], axis=1)
    t2 = jnp.concatenate([z, x_f[:, 2 * w34:], z], axis=1)
    x3 = jnp.concatenate([t0, x_f, t2], axis=0)
    y1 = jnp.dot(w1c, x3,
                 preferred_element_type=jnp.float32).astype(bf16)
    u0 = jnp.concatenate([y1[:, ell - 1:], y1[:, :ell - 1]], axis=1)
    u2 = jnp.concatenate([y1[:, 1:], y1[:, :1]], axis=1)
    y3 = jnp.concatenate([u0, y1, u2], axis=0)
    return jnp.dot(w2c, y3, preferred_element_type=jnp.float32)


def _stats_kernel(x_ref, w1c_ref, w2c_ref, psum_ref, psq_ref, *, dims):
    c_in, c_mid, w34, ell = dims
    acc2 = _conv_core(x_ref[0], w1c_ref[...], w2c_ref[...],
                      c_in, c_mid, w34, ell)
    psum_ref[0] = jnp.sum(acc2, axis=1, keepdims=True)
    psq_ref[0] = jnp.sum(acc2 * acc2, axis=1, keepdims=True)


def _out_kernel(x_ref, w1c_ref, w2s_ref, bias_ref, o_ref, *, dims):
    c_in, c_mid, w34, ell = dims
    acc2 = _conv_core(x_ref[0], w1c_ref[...], w2s_ref[...],
                      c_in, c_mid, w34, ell)
    o_ref[0] = (acc2 + bias_ref[...]).astype(o_ref.dtype)


def kernel(x, w1, w2):
    n, c_in, h, w = x.shape
    c_mid = w1.shape[0]
    c_out = w2.shape[0]
    k = 3
    eps = 1e-5
    h2 = h + 2
    w34 = w + 2
    ell = h2 * w34
    dims = (c_in, c_mid, w34, ell)

    f32 = jnp.float32
    bf16 = jnp.bfloat16
    w1c = jnp.concatenate([w1[:, :, t, 0] for t in range(k)],
                          axis=1).astype(bf16)
    w2c_f = jnp.concatenate([w2[:, :, 0, t] for t in range(k)],
                            axis=1).astype(f32)

    psum, psq = pl.pallas_call(
        functools.partial(_stats_kernel, dims=dims),
        out_shape=(
            jax.ShapeDtypeStruct((n, c_out, 1), f32),
            jax.ShapeDtypeStruct((n, c_out, 1), f32),
        ),
        grid=(n,),
        in_specs=[
            pl.BlockSpec((1, c_in, h, w), lambda i: (i, 0, 0, 0)),
            pl.BlockSpec((c_mid, k * c_in), lambda i: (0, 0)),
            pl.BlockSpec((c_out, k * c_mid), lambda i: (0, 0)),
        ],
        out_specs=(
            pl.BlockSpec((1, c_out, 1), lambda i: (i, 0, 0)),
            pl.BlockSpec((1, c_out, 1), lambda i: (i, 0, 0)),
        ),
        compiler_params=pltpu.CompilerParams(
            dimension_semantics=("parallel",)),
    )(x, w1c, w2c_f.astype(bf16))

    cnt = jnp.float32(n * ell)
    mean = jnp.sum(psum, axis=0) / cnt
    var = jnp.sum(psq, axis=0) / cnt - mean * mean
    rstd = lax.rsqrt(var + eps)
    w2s = (w2c_f * rstd).astype(bf16)
    bias = (-mean * rstd)

    out_flat = pl.pallas_call(
        functools.partial(_out_kernel, dims=dims),
        out_shape=jax.ShapeDtypeStruct((n, c_out, ell), x.dtype),
        grid=(n,),
        in_specs=[
            pl.BlockSpec((1, c_in, h, w), lambda i: (i, 0, 0, 0)),
            pl.BlockSpec((c_mid, k * c_in), lambda i: (0, 0)),
            pl.BlockSpec((c_out, k * c_mid), lambda i: (0, 0)),
            pl.BlockSpec((c_out, 1), lambda i: (0, 0)),
        ],
        out_specs=pl.BlockSpec((1, c_out, ell), lambda i: (i, 0, 0)),
        compiler_params=pltpu.CompilerParams(
            dimension_semantics=("parallel",)),
    )(x, w1c, w2s, bias)

    return out_flat.reshape(n, c_out, h2, w34)


# speedup vs baseline: 1.8039x; 1.0678x over previous
import functools

import jax
import jax.numpy as jnp
from jax import lax
from jax.experimental import pallas as pl
from jax.experimental.pallas import tpu as pltpu


def _conv_core(x_blk, w1c, w2c, c_in, c_mid, w34, ell):
    bf16 = jnp.bfloat16
    xr = jnp.maximum(x_blk, 0.0).astype(bf16)
    xpad = jnp.pad(xr, ((0, 0), (1, 1), (1, 1)))
    x_f = xpad.reshape(c_in, ell)
    z = jnp.zeros((c_in, w34), bf16)
    span = ell - 2 * w34
    t0 = jnp.concatenate([z, x_f[:, :span], z], axis=1)
    t2 = jnp.concatenate([z, x_f[:, 2 * w34:], z], axis=1)
    x3 = jnp.concatenate([t0, x_f, t2], axis=0)
    y1 = jnp.dot(w1c, x3,
                 preferred_element_type=jnp.float32).astype(bf16)
    u0 = jnp.concatenate([y1[:, ell - 1:], y1[:, :ell - 1]], axis=1)
    u2 = jnp.concatenate([y1[:, 1:], y1[:, :1]], axis=1)
    y3 = jnp.concatenate([u0, y1, u2], axis=0)
    return jnp.dot(w2c, y3, preferred_element_type=jnp.float32)


def _stats_kernel(x_ref, w1c_ref, w2c_ref, psum_ref, psq_ref, *, dims, bimg):
    c_in, c_mid, w34, ell = dims
    for b in range(bimg):
        acc2 = _conv_core(x_ref[b], w1c_ref[...], w2c_ref[...],
                          c_in, c_mid, w34, ell)
        psum_ref[b] = jnp.sum(acc2, axis=1, keepdims=True)
        psq_ref[b] = jnp.sum(acc2 * acc2, axis=1, keepdims=True)


def _out_kernel(x_ref, w1c_ref, w2s_ref, bias_ref, o_ref, *, dims, bimg):
    c_in, c_mid, w34, ell = dims
    for b in range(bimg):
        acc2 = _conv_core(x_ref[b], w1c_ref[...], w2s_ref[...],
                          c_in, c_mid, w34, ell)
        o_ref[b] = (acc2 + bias_ref[...]).astype(o_ref.dtype)


def kernel(x, w1, w2):
    n, c_in, h, w = x.shape
    c_mid = w1.shape[0]
    c_out = w2.shape[0]
    k = 3
    eps = 1e-5
    h2 = h + 2
    w34 = w + 2
    ell = h2 * w34
    dims = (c_in, c_mid, w34, ell)

    f32 = jnp.float32
    bf16 = jnp.bfloat16
    w1c = jnp.concatenate([w1[:, :, t, 0] for t in range(k)],
                          axis=1).astype(bf16)
    w2c_f = jnp.concatenate([w2[:, :, 0, t] for t in range(k)],
                            axis=1).astype(f32)

    bimg = 8 if n % 8 == 0 else 1

    psum, psq = pl.pallas_call(
        functools.partial(_stats_kernel, dims=dims, bimg=bimg),
        out_shape=(
            jax.ShapeDtypeStruct((n, c_out, 1), f32),
            jax.ShapeDtypeStruct((n, c_out, 1), f32),
        ),
        grid=(n // bimg,),
        in_specs=[
            pl.BlockSpec((bimg, c_in, h, w), lambda i: (i, 0, 0, 0)),
            pl.BlockSpec((c_mid, k * c_in), lambda i: (0, 0)),
            pl.BlockSpec((c_out, k * c_mid), lambda i: (0, 0)),
        ],
        out_specs=(
            pl.BlockSpec((bimg, c_out, 1), lambda i: (i, 0, 0)),
            pl.BlockSpec((bimg, c_out, 1), lambda i: (i, 0, 0)),
        ),
        compiler_params=pltpu.CompilerParams(
            dimension_semantics=("parallel",)),
    )(x, w1c, w2c_f.astype(bf16))

    cnt = jnp.float32(n * ell)
    mean = jnp.sum(psum, axis=0) / cnt
    var = jnp.sum(psq, axis=0) / cnt - mean * mean
    rstd = lax.rsqrt(var + eps)
    w2s = (w2c_f * rstd).astype(bf16)
    bias = (-mean * rstd)

    out_flat = pl.pallas_call(
        functools.partial(_out_kernel, dims=dims, bimg=bimg),
        out_shape=jax.ShapeDtypeStruct((n, c_out, ell), x.dtype),
        grid=(n // bimg,),
        in_specs=[
            pl.BlockSpec((bimg, c_in, h, w), lambda i: (i, 0, 0, 0)),
            pl.BlockSpec((c_mid, k * c_in), lambda i: (0, 0)),
            pl.BlockSpec((c_out, k * c_mid), lambda i: (0, 0)),
            pl.BlockSpec((c_out, 1), lambda i: (0, 0)),
        ],
        out_specs=pl.BlockSpec((bimg, c_out, ell), lambda i: (i, 0, 0)),
        compiler_params=pltpu.CompilerParams(
            dimension_semantics=("parallel",)),
    )(x, w1c, w2s, bias)

    return out_flat.reshape(n, c_out, h2, w34)


# speedup vs baseline: 2.1052x; 1.1670x over previous
import functools

import jax
import jax.numpy as jnp
from jax import lax
from jax.experimental import pallas as pl
from jax.experimental.pallas import tpu as pltpu


def _prep_x(x_blk, c_in, ell):
    xr = jnp.maximum(x_blk.astype(jnp.bfloat16), 0)
    xpad = jnp.pad(xr, ((0, 0), (1, 1), (1, 1)))
    return xpad.reshape(c_in, ell)


def _conv_core(x_f, w1c, w2c, c_in, c_mid, w34, ell):
    bf16 = jnp.bfloat16
    z = jnp.zeros((c_in, w34), bf16)
    span = ell - 2 * w34
    t0 = jnp.concatenate([z, x_f[:, :span], z], axis=1)
    t2 = jnp.concatenate([z, x_f[:, 2 * w34:], z], axis=1)
    x3 = jnp.concatenate([t0, x_f, t2], axis=0)
    y1 = jnp.dot(w1c, x3,
                 preferred_element_type=jnp.float32).astype(bf16)
    u0 = jnp.concatenate([y1[:, ell - 1:], y1[:, :ell - 1]], axis=1)
    u2 = jnp.concatenate([y1[:, 1:], y1[:, :1]], axis=1)
    y3 = jnp.concatenate([u0, y1, u2], axis=0)
    return jnp.dot(w2c, y3, preferred_element_type=jnp.float32)


def _stats_kernel(x_ref, w1c_ref, w2c_ref, psum_ref, psq_ref, xp_ref, *,
                  dims, bimg):
    c_in, c_mid, w34, ell = dims
    for b in range(bimg):
        x_f = _prep_x(x_ref[b], c_in, ell)
        xp_ref[b] = x_f
        acc2 = _conv_core(x_f, w1c_ref[...], w2c_ref[...],
                          c_in, c_mid, w34, ell)
        psum_ref[b] = jnp.sum(acc2, axis=1, keepdims=True)
        psq_ref[b] = jnp.sum(acc2 * acc2, axis=1, keepdims=True)


def _out_kernel(xp_ref, w1c_ref, w2s_ref, bias_ref, o_ref, *, dims, bimg):
    c_in, c_mid, w34, ell = dims
    for b in range(bimg):
        acc2 = _conv_core(xp_ref[b], w1c_ref[...], w2s_ref[...],
                          c_in, c_mid, w34, ell)
        o_ref[b] = (acc2 + bias_ref[...]).astype(o_ref.dtype)


def kernel(x, w1, w2):
    n, c_in, h, w = x.shape
    c_mid = w1.shape[0]
    c_out = w2.shape[0]
    k = 3
    eps = 1e-5
    h2 = h + 2
    w34 = w + 2
    ell = h2 * w34
    dims = (c_in, c_mid, w34, ell)

    f32 = jnp.float32
    bf16 = jnp.bfloat16
    w1c = jnp.concatenate([w1[:, :, t, 0] for t in range(k)],
                          axis=1).astype(bf16)
    w2c_f = jnp.concatenate([w2[:, :, 0, t] for t in range(k)],
                            axis=1).astype(f32)

    bimg = 8 if n % 8 == 0 else 1

    psum, psq, xprep = pl.pallas_call(
        functools.partial(_stats_kernel, dims=dims, bimg=bimg),
        out_shape=(
            jax.ShapeDtypeStruct((n, c_out, 1), f32),
            jax.ShapeDtypeStruct((n, c_out, 1), f32),
            jax.ShapeDtypeStruct((n, c_in, ell), bf16),
        ),
        grid=(n // bimg,),
        in_specs=[
            pl.BlockSpec((bimg, c_in, h, w), lambda i: (i, 0, 0, 0)),
            pl.BlockSpec((c_mid, k * c_in), lambda i: (0, 0)),
            pl.BlockSpec((c_out, k * c_mid), lambda i: (0, 0)),
        ],
        out_specs=(
            pl.BlockSpec((bimg, c_out, 1), lambda i: (i, 0, 0)),
            pl.BlockSpec((bimg, c_out, 1), lambda i: (i, 0, 0)),
            pl.BlockSpec((bimg, c_in, ell), lambda i: (i, 0, 0)),
        ),
        compiler_params=pltpu.CompilerParams(
            dimension_semantics=("parallel",)),
    )(x, w1c, w2c_f.astype(bf16))

    cnt = jnp.float32(n * ell)
    mean = jnp.sum(psum, axis=0) / cnt
    var = jnp.sum(psq, axis=0) / cnt - mean * mean
    rstd = lax.rsqrt(var + eps)
    w2s = (w2c_f * rstd).astype(bf16)
    bias = (-mean * rstd)

    out_flat = pl.pallas_call(
        functools.partial(_out_kernel, dims=dims, bimg=bimg),
        out_shape=jax.ShapeDtypeStruct((n, c_out, ell), x.dtype),
        grid=(n // bimg,),
        in_specs=[
            pl.BlockSpec((bimg, c_in, ell), lambda i: (i, 0, 0)),
            pl.BlockSpec((c_mid, k * c_in), lambda i: (0, 0)),
            pl.BlockSpec((c_out, k * c_mid), lambda i: (0, 0)),
            pl.BlockSpec((c_out, 1), lambda i: (0, 0)),
        ],
        out_specs=pl.BlockSpec((bimg, c_out, ell), lambda i: (i, 0, 0)),
        compiler_params=pltpu.CompilerParams(
            dimension_semantics=("parallel",)),
    )(xprep, w1c, w2s, bias)

    return out_flat.reshape(n, c_out, h2, w34)
